# baseline (speedup 1.0000x reference)
import functools

import jax
import jax.numpy as jnp
from jax import lax
from jax.experimental import pallas as pl

NQ_ = 6
K_ = 1024
D_ = 256
CHUNK = 1024


def _rvq_body(x_ref, embs_ref, ehi_ref, emid_ref, elo_ref, q_ref, tok_ref,
              loss_ref):
    x = x_ref[...]
    r = x
    qsum = None

    col_iota_k = lax.broadcasted_iota(jnp.int32, (CHUNK, K_), 1)
    tok_cols = lax.broadcasted_iota(jnp.int32, (CHUNK, 8), 1)
    lrow = lax.broadcasted_iota(jnp.int32, (8, 128), 0)
    lcol = lax.broadcasted_iota(jnp.int32, (8, 128), 1)

    tok_acc = jnp.zeros((CHUNK, 8), jnp.int32)
    lacc = jnp.zeros((8, 128), jnp.float32)

    for v in range(NQ_):
        emb = embs_ref[v]
        e2 = jnp.sum(emb * emb, axis=0, keepdims=True)
        r2 = jnp.sum(r * r, axis=1, keepdims=True)
        xe = jnp.dot(r, emb, preferred_element_type=jnp.float32,
                     precision=lax.Precision.DEFAULT)
        dist = r2 + e2 - 2.0 * xe
        dmin = jnp.min(dist, axis=1, keepdims=True)
        idx = jnp.min(jnp.where(dist == dmin, col_iota_k, K_), axis=1,
                      keepdims=True)
        onehot = (col_iota_k == idx).astype(jnp.float32)
        dn = (((1,), (1,)), ((), ()))
        q = (lax.dot_general(onehot, ehi_ref[v], dn,
                             preferred_element_type=jnp.float32,
                             precision=lax.Precision.DEFAULT)
             + lax.dot_general(onehot, emid_ref[v], dn,
                               preferred_element_type=jnp.float32,
                               precision=lax.Precision.DEFAULT)
             + lax.dot_general(onehot, elo_ref[v], dn,
                               preferred_element_type=jnp.float32,
                               precision=lax.Precision.DEFAULT))
        rn = r - q
        qsum = q if qsum is None else qsum + q
        lacc = lacc + jnp.sum(rn * rn) * ((lrow == v) & (lcol == 0)).astype(jnp.float32)
        tok_acc = tok_acc + idx * (tok_cols == v).astype(jnp.int32)
        r = rn

    q_ref[...] = qsum
    tok_ref[0] = tok_acc
    loss_ref[0] = lacc


@jax.jit
def kernel(inputs, embs):
    B, D, N = inputs.shape
    flat = jnp.transpose(inputs, (0, 2, 1)).reshape(B * N, D)
    nsteps = (B * N) // CHUNK

    mask = jnp.int32(-65536)

    def _trunc16(a):
        return lax.bitcast_convert_type(
            lax.bitcast_convert_type(a, jnp.int32) & mask, jnp.float32)

    ehi = _trunc16(embs)
    rem = embs - ehi
    emid = _trunc16(rem)
    elo = rem - emid

    full_emb = pl.BlockSpec((NQ_, D_, K_), lambda i: (0, 0, 0))
    qsum, tok, lpart = pl.pallas_call(
        _rvq_body,
        grid=(nsteps,),
        in_specs=[
            pl.BlockSpec((CHUNK, D_), lambda i: (i, 0)),
            full_emb, full_emb, full_emb, full_emb,
        ],
        out_specs=[
            pl.BlockSpec((CHUNK, D_), lambda i: (i, 0)),
            pl.BlockSpec((1, CHUNK, 8), lambda i: (i, 0, 0)),
            pl.BlockSpec((1, 8, 128), lambda i: (i, 0, 0)),
        ],
        out_shape=[
            jax.ShapeDtypeStruct((B * N, D_), jnp.float32),
            jax.ShapeDtypeStruct((nsteps, CHUNK, 8), jnp.int32),
            jax.ShapeDtypeStruct((nsteps, 8, 128), jnp.float32),
        ],
    )(flat, embs, ehi, emid, elo)

    quantized = jnp.transpose(qsum.reshape(B, N, D), (0, 2, 1))
    tokens = jnp.transpose(tok[:, :, :NQ_], (2, 0, 1))
    layer_sums = jnp.sum(lpart[:, :NQ_, 0], axis=0)
    loss = jnp.sum(layer_sums / jnp.float32(B * D * N))
    return quantized, tokens, loss


# speedup vs baseline: 1.4149x; 1.4149x over previous
import functools

import jax
import jax.numpy as jnp
from jax import lax
from jax.experimental import pallas as pl
from jax.experimental.pallas import tpu as pltpu

NQ_ = 6
K_ = 1024
D_ = 256
CHUNK = 1024
NS_ = 2
SROWS = CHUNK // NS_


def _rvq_body(x_ref, embs_ref, ehi_ref, emid_ref, elo_ref, q_ref, tok_ref,
              loss_ref, e2_scr):
    @pl.when(pl.program_id(0) == 0)
    def _():
        for v in range(NQ_):
            emb = embs_ref[v]
            e2_scr[v] = jnp.sum(emb * emb, axis=0, keepdims=True)

    col_iota_k = lax.broadcasted_iota(jnp.int32, (SROWS, K_), 1)
    tok_cols = lax.broadcasted_iota(jnp.int32, (SROWS, 8), 1)
    lrow = lax.broadcasted_iota(jnp.int32, (8, 128), 0)
    lcol = lax.broadcasted_iota(jnp.int32, (8, 128), 1)
    dn = (((1,), (1,)), ((), ()))

    r = [x_ref[pl.ds(h * SROWS, SROWS), :] for h in range(NS_)]
    qsum = [None] * NS_
    tok_acc = [jnp.zeros((SROWS, 8), jnp.int32) for _ in range(NS_)]
    lacc = jnp.zeros((8, 128), jnp.float32)

    for v in range(NQ_):
        emb = embs_ref[v]
        e2 = e2_scr[v]
        lmask = ((lrow == v) & (lcol == 0)).astype(jnp.float32)
        for h in range(NS_):
            r2 = jnp.sum(r[h] * r[h], axis=1, keepdims=True)
            xe = jnp.dot(r[h], emb, preferred_element_type=jnp.float32,
                         precision=lax.Precision.DEFAULT)
            dist = r2 + e2 - 2.0 * xe
            dmin = jnp.min(dist, axis=1, keepdims=True)
            idx = jnp.min(jnp.where(dist == dmin, col_iota_k, K_), axis=1,
                          keepdims=True)
            onehot = (col_iota_k == idx).astype(jnp.float32)
            q = (lax.dot_general(onehot, ehi_ref[v], dn,
                                 preferred_element_type=jnp.float32,
                                 precision=lax.Precision.DEFAULT)
                 + lax.dot_general(onehot, emid_ref[v], dn,
                                   preferred_element_type=jnp.float32,
                                   precision=lax.Precision.DEFAULT)
                 + lax.dot_general(onehot, elo_ref[v], dn,
                                   preferred_element_type=jnp.float32,
                                   precision=lax.Precision.DEFAULT))
            rn = r[h] - q
            qsum[h] = q if qsum[h] is None else qsum[h] + q
            lacc = lacc + jnp.sum(rn * rn) * lmask
            tok_acc[h] = tok_acc[h] + idx * (tok_cols == v).astype(jnp.int32)
            r[h] = rn

    for h in range(NS_):
        q_ref[pl.ds(h * SROWS, SROWS), :] = qsum[h]
        tok_ref[0, pl.ds(h * SROWS, SROWS), :] = tok_acc[h]
    loss_ref[0] = lacc


@jax.jit
def kernel(inputs, embs):
    B, D, N = inputs.shape
    flat = jnp.transpose(inputs, (0, 2, 1)).reshape(B * N, D)
    nsteps = (B * N) // CHUNK

    mask = jnp.int32(-65536)

    def _trunc16(a):
        return lax.bitcast_convert_type(
            lax.bitcast_convert_type(a, jnp.int32) & mask, jnp.float32)

    ehi = _trunc16(embs)
    rem = embs - ehi
    emid = _trunc16(rem)
    elo = rem - emid

    full_emb = pl.BlockSpec((NQ_, D_, K_), lambda i: (0, 0, 0))
    qsum, tok, lpart = pl.pallas_call(
        _rvq_body,
        grid=(nsteps,),
        in_specs=[
            pl.BlockSpec((CHUNK, D_), lambda i: (i, 0)),
            full_emb, full_emb, full_emb, full_emb,
        ],
        out_specs=[
            pl.BlockSpec((CHUNK, D_), lambda i: (i, 0)),
            pl.BlockSpec((1, CHUNK, 8), lambda i: (i, 0, 0)),
            pl.BlockSpec((1, 8, 128), lambda i: (i, 0, 0)),
        ],
        out_shape=[
            jax.ShapeDtypeStruct((B * N, D_), jnp.float32),
            jax.ShapeDtypeStruct((nsteps, CHUNK, 8), jnp.int32),
            jax.ShapeDtypeStruct((nsteps, 8, 128), jnp.float32),
        ],
        scratch_shapes=[pltpu.VMEM((NQ_, 1, K_), jnp.float32)],
    )(flat, embs, ehi, emid, elo)

    quantized = jnp.transpose(qsum.reshape(B, N, D), (0, 2, 1))
    tokens = jnp.transpose(tok[:, :, :NQ_], (2, 0, 1))
    layer_sums = jnp.sum(lpart[:, :NQ_, 0], axis=0)
    loss = jnp.sum(layer_sums / jnp.float32(B * D * N))
    return quantized, tokens, loss
